# baseline (speedup 1.0000x reference)
import functools

import jax
import jax.numpy as jnp
from jax import lax
from jax.experimental import pallas as pl
from jax.experimental.pallas import tpu as pltpu
from jax.experimental.pallas import tpu_sc as plsc

T_ROWS = 50015
EMB = 128
N_DET = 20000
DET_CHUNK = 2000
N_PART = 16
ROWS_PER = 3136
N_ITEMS_GAME = 28672
ITEM_D = 64
ITEM_PER_W = N_ITEMS_GAME // 32

OUT_N = 512 + 2 * T_ROWS * EMB + N_ITEMS_GAME * ITEM_D
SO_ROW0 = 4
MO_ROW0 = SO_ROW0 + T_ROWS
IT_ROW64 = (512 + 2 * T_ROWS * EMB) // ITEM_D
TAIL = T_ROWS - 15 * ROWS_PER


def _scatter_one_table(det_hbm, bufs, sems, claim_v, tab_v, width, ncols, lo):
    lanes = lax.iota(jnp.int32, 16)
    nch = N_DET // DET_CHUNK

    copies = [None, None]
    copies[0] = pltpu.async_copy(det_hbm.at[pl.ds(0, DET_CHUNK * width)],
                                 bufs[0].at[pl.ds(0, DET_CHUNK * width)],
                                 sems[0])
    for ci in range(nch):
        sl = ci % 2
        copies[sl].wait()
        if ci + 1 < nch:
            nsl = (ci + 1) % 2
            copies[nsl] = pltpu.async_copy(
                det_hbm.at[pl.ds((ci + 1) * DET_CHUNK * width,
                                 DET_CHUNK * width)],
                bufs[nsl].at[pl.ds(0, DET_CHUNK * width)], sems[nsl])
        det_v = bufs[sl]
        c0 = ci * DET_CHUNK

        def vreg_body(v, _):
            row = v * 16 + lanes
            j = c0 + row
            flat = row * width
            sid = plsc.load_gather(det_v, [flat])
            m = (sid >= lo) & (sid < lo + ROWS_PER)
            rel = jnp.where(m, sid - lo, 0)

            plsc.store_scatter(claim_v, [rel], j, mask=m)
            cc0 = plsc.load_gather(claim_v, [rel])

            def wcond(cc):
                return jnp.any(m & (j > cc))

            def wbody(cc):
                plsc.store_scatter(claim_v, [rel], j, mask=m & (j > cc))
                return plsc.load_gather(claim_v, [rel])

            ccf = lax.while_loop(wcond, wbody, cc0)
            win = m & (ccf == j)
            for k in range(ncols):
                dk = plsc.load_gather(det_v, [flat + (k + 1)])
                plsc.store_scatter(tab_v, [rel + k * ROWS_PER],
                                   dk.astype(jnp.float32), mask=win)
            return 0

        lax.fori_loop(0, DET_CHUNK // 16, vreg_body, 0)


def _sc_irregular(screen_det, minimap_det, items, item_emb):
    mesh = plsc.VectorSubcoreMesh(core_axis_name="c", subcore_axis_name="s")

    @functools.partial(
        pl.kernel,
        out_type=(
            jax.ShapeDtypeStruct((N_PART, 4, ROWS_PER), jnp.float32),
            jax.ShapeDtypeStruct((N_PART, 2, ROWS_PER), jnp.float32),
            jax.ShapeDtypeStruct((N_ITEMS_GAME, ITEM_D), jnp.float32),
        ),
        mesh=mesh,
        scratch_types=[
            pltpu.VMEM((DET_CHUNK * 5,), jnp.int32),
            pltpu.VMEM((DET_CHUNK * 5,), jnp.int32),
            pltpu.VMEM((ROWS_PER,), jnp.int32),
            pltpu.VMEM((4 * ROWS_PER,), jnp.float32),
            pltpu.VMEM((ITEM_PER_W,), jnp.int32),
            pltpu.VMEM((ITEM_PER_W, ITEM_D), jnp.float32),
            pltpu.SemaphoreType.DMA,
            pltpu.SemaphoreType.DMA,
            pltpu.SemaphoreType.DMA,
        ],
        compiler_params=pltpu.CompilerParams(needs_layout_passes=False,
                                             use_tc_tiling_on_sc=False),
    )
    def sc_kernel(sdet_hbm, mdet_hbm, items_hbm, emb_hbm,
                  sfT_hbm, mfT_hbm, irows_hbm,
                  deta_v, detb_v, claim_v, tab_v, idx_v, rows_v,
                  sema, semb, semg):
        c = lax.axis_index("c")
        s = lax.axis_index("s")
        wid = s * 2 + c

        ibase = wid * ITEM_PER_W
        pltpu.sync_copy(items_hbm.at[pl.ds(ibase, ITEM_PER_W)], idx_v)
        gcopy = pltpu.async_copy(emb_hbm.at[idx_v], rows_v, semg)

        part = wid & 15
        lo = part * ROWS_PER

        def claim_init(i, _):
            claim_v[pl.ds(i * 16, 16)] = jnp.full((16,), -1, jnp.int32)
            return 0
        lax.fori_loop(0, ROWS_PER // 16, claim_init, 0)

        def tab_init(i, _):
            tab_v[pl.ds(i * 16, 16)] = jnp.zeros((16,), jnp.float32)
            return 0
        lax.fori_loop(0, 4 * ROWS_PER // 16, tab_init, 0)

        @pl.when(wid < 16)
        def _():
            _scatter_one_table(sdet_hbm, (deta_v, detb_v), (sema, semb),
                               claim_v, tab_v, 5, 4, lo)
            for k in range(4):
                pltpu.sync_copy(tab_v.at[pl.ds(k * ROWS_PER, ROWS_PER)],
                                sfT_hbm.at[part, k])

        @pl.when(wid >= 16)
        def _():
            _scatter_one_table(mdet_hbm, (deta_v, detb_v), (sema, semb),
                               claim_v, tab_v, 3, 2, lo)
            for k in range(2):
                pltpu.sync_copy(tab_v.at[pl.ds(k * ROWS_PER, ROWS_PER)],
                                mfT_hbm.at[part, k])

        gcopy.wait()
        pltpu.sync_copy(rows_v, irows_hbm.at[pl.ds(ibase, ITEM_PER_W)])

    return sc_kernel(screen_det.reshape(-1), minimap_det.reshape(-1),
                     items, item_emb)


def _tc_mlps(char_emb, sfT, mfT, A1s, B1s, bs1, W2s, bs2, A1m, B1m, bm1, W2m, bm2):
    def body(char_ref, sfT_ref, mfT_ref,
             a1s_ref, b1s_ref, bs1_ref, w2s_ref, bs2_ref,
             a1m_ref, b1m_ref, bm1_ref, w2m_ref, bm2_ref,
             so_ref, mo_ref):
        cb = char_ref[...]
        sft = sfT_ref[0]
        mft = mfT_ref[0]
        dn = (((0,), (0,)), ((), ()))
        cs = lax.dot_general(sft, b1s_ref[...], dn,
                             preferred_element_type=jnp.float32)
        h = jnp.dot(cb, a1s_ref[...], preferred_element_type=jnp.float32)
        h = jnp.maximum(h + cs + bs1_ref[...], 0.0)
        so = jnp.dot(h, w2s_ref[...], preferred_element_type=jnp.float32)
        so_ref[...] = (so + bs2_ref[...])[None]
        cm = lax.dot_general(mft, b1m_ref[...], dn,
                             preferred_element_type=jnp.float32)
        hm = jnp.dot(cb, a1m_ref[...], preferred_element_type=jnp.float32)
        hm = jnp.maximum(hm + cm + bm1_ref[...], 0.0)
        mo = jnp.dot(hm, w2m_ref[...], preferred_element_type=jnp.float32)
        mo_ref[...] = (mo + bm2_ref[...])[None]

    full = lambda shape: pl.BlockSpec(shape, lambda i: (0,) * len(shape))
    return pl.pallas_call(
        body,
        grid=(N_PART,),
        in_specs=[
            pl.BlockSpec((ROWS_PER, EMB), lambda i: (i, 0)),
            pl.BlockSpec((1, 4, ROWS_PER), lambda i: (i, 0, 0)),
            pl.BlockSpec((1, 2, ROWS_PER), lambda i: (i, 0, 0)),
            full((EMB, EMB)), full((4, EMB)), full((1, EMB)), full((EMB, EMB)), full((1, EMB)),
            full((EMB, EMB)), full((2, EMB)), full((1, EMB)), full((EMB, EMB)), full((1, EMB)),
        ],
        out_specs=[
            pl.BlockSpec((1, ROWS_PER, EMB), lambda i: (i, 0, 0)),
            pl.BlockSpec((1, ROWS_PER, EMB), lambda i: (i, 0, 0)),
        ],
        out_shape=[
            jax.ShapeDtypeStruct((N_PART, ROWS_PER, EMB), jnp.float32),
            jax.ShapeDtypeStruct((N_PART, ROWS_PER, EMB), jnp.float32),
        ],
    )(char_emb, sfT, mfT, A1s, B1s, bs1, W2s, bs2, A1m, B1m, bm1, W2m, bm2)


_SO_LEN = T_ROWS * EMB
_IT_LEN = N_ITEMS_GAME * ITEM_D
_N_CHUNK = 5


def _tc_assemble(cont, so1d, mo1d, ir1d):
    def body(cont_ref, so_ref, mo_ref, ir_ref, out_ref, sem):
        copies = [pltpu.async_copy(cont_ref, out_ref.at[0, pl.ds(0, 512)], sem)]
        cs = _SO_LEN // _N_CHUNK
        for k in range(_N_CHUNK):
            copies.append(pltpu.async_copy(
                so_ref.at[pl.ds(k * cs, cs)],
                out_ref.at[0, pl.ds(512 + k * cs, cs)], sem))
            copies.append(pltpu.async_copy(
                mo_ref.at[pl.ds(k * cs, cs)],
                out_ref.at[0, pl.ds(512 + _SO_LEN + k * cs, cs)], sem))
        ci = _IT_LEN // 4
        for k in range(4):
            copies.append(pltpu.async_copy(
                ir_ref.at[pl.ds(k * ci, ci)],
                out_ref.at[0, pl.ds(512 + 2 * _SO_LEN + k * ci, ci)], sem))
        for c in copies:
            c.wait()

    anyspec = pl.BlockSpec(memory_space=pl.ANY)
    return pl.pallas_call(
        body,
        in_specs=[anyspec, anyspec, anyspec, anyspec],
        out_specs=pl.BlockSpec(memory_space=pl.ANY),
        out_shape=jax.ShapeDtypeStruct((1, OUT_N), jnp.float32),
        scratch_shapes=[pltpu.SemaphoreType.DMA],
    )(cont, so1d, mo1d, ir1d)


def kernel(continuous_f, screen_detections, minimap_detections, items, char_emb,
           item_emb, Ws1, bs1, Ws2, bs2, Wm1, bm1, Wm2, bm2):
    screen_detections = screen_detections.astype(jnp.int32)
    minimap_detections = minimap_detections.astype(jnp.int32)
    items = items.astype(jnp.int32)

    sfT, mfT, irows = _sc_irregular(screen_detections, minimap_detections,
                                    items, item_emb)

    A1s = Ws1[:, :EMB].T
    B1s = Ws1[:, EMB:].T
    W2s = Ws2.T
    A1m = Wm1[:, :EMB].T
    B1m = Wm1[:, EMB:].T
    W2m = Wm2.T

    so3, mo3 = _tc_mlps(
        char_emb, sfT, mfT,
        A1s, B1s, bs1.reshape(1, EMB), W2s, bs2.reshape(1, EMB),
        A1m, B1m, bm1.reshape(1, EMB), W2m, bm2.reshape(1, EMB))

    return _tc_assemble(continuous_f, so3.reshape(-1), mo3.reshape(-1),
                        irows.reshape(-1))


# speedup vs baseline: 7.4352x; 7.4352x over previous
import functools

import jax
import jax.numpy as jnp
from jax import lax
from jax.experimental import pallas as pl
from jax.experimental.pallas import tpu as pltpu
from jax.experimental.pallas import tpu_sc as plsc

T_ROWS = 50015
EMB = 128
N_DET = 20000
DET_CHUNK = 2000
N_PART = 16
ROWS_PER = 3136
N_ITEMS_GAME = 28672
ITEM_D = 64
ITEM_PER_W = N_ITEMS_GAME // 32

OUT_N = 512 + 2 * T_ROWS * EMB + N_ITEMS_GAME * ITEM_D
SO_ROW0 = 4
MO_ROW0 = SO_ROW0 + T_ROWS
IT_ROW64 = (512 + 2 * T_ROWS * EMB) // ITEM_D
TAIL = T_ROWS - 15 * ROWS_PER


def _scatter_one_table(det_hbm, bufs, sems, claim_v, tab_v, width, ncols, lo):
    lanes = lax.iota(jnp.int32, 16)
    nch = N_DET // DET_CHUNK

    copies = [None, None]
    copies[0] = pltpu.async_copy(det_hbm.at[pl.ds(0, DET_CHUNK * width)],
                                 bufs[0].at[pl.ds(0, DET_CHUNK * width)],
                                 sems[0])
    for ci in range(nch):
        sl = ci % 2
        copies[sl].wait()
        if ci + 1 < nch:
            nsl = (ci + 1) % 2
            copies[nsl] = pltpu.async_copy(
                det_hbm.at[pl.ds((ci + 1) * DET_CHUNK * width,
                                 DET_CHUNK * width)],
                bufs[nsl].at[pl.ds(0, DET_CHUNK * width)], sems[nsl])
        det_v = bufs[sl]
        c0 = ci * DET_CHUNK

        def vreg_body(v, _):
            row = v * 16 + lanes
            j = c0 + row
            flat = row * width
            sid = plsc.load_gather(det_v, [flat])
            m = (sid >= lo) & (sid < lo + ROWS_PER)
            rel = jnp.where(m, sid - lo, 0)

            plsc.store_scatter(claim_v, [rel], j, mask=m)
            cc0 = plsc.load_gather(claim_v, [rel])

            def wcond(cc):
                return jnp.any(m & (j > cc))

            def wbody(cc):
                plsc.store_scatter(claim_v, [rel], j, mask=m & (j > cc))
                return plsc.load_gather(claim_v, [rel])

            ccf = lax.while_loop(wcond, wbody, cc0)
            win = m & (ccf == j)
            for k in range(ncols):
                dk = plsc.load_gather(det_v, [flat + (k + 1)])
                plsc.store_scatter(tab_v, [rel + k * ROWS_PER],
                                   dk.astype(jnp.float32), mask=win)
            return 0

        lax.fori_loop(0, DET_CHUNK // 16, vreg_body, 0)


def _sc_irregular(screen_det, minimap_det, items, item_emb):
    mesh = plsc.VectorSubcoreMesh(core_axis_name="c", subcore_axis_name="s")

    @functools.partial(
        pl.kernel,
        out_type=(
            jax.ShapeDtypeStruct((N_PART, 4, ROWS_PER), jnp.float32),
            jax.ShapeDtypeStruct((N_PART, 2, ROWS_PER), jnp.float32),
            jax.ShapeDtypeStruct((N_ITEMS_GAME, ITEM_D), jnp.float32),
        ),
        mesh=mesh,
        scratch_types=[
            pltpu.VMEM((DET_CHUNK * 5,), jnp.int32),
            pltpu.VMEM((DET_CHUNK * 5,), jnp.int32),
            pltpu.VMEM((ROWS_PER,), jnp.int32),
            pltpu.VMEM((4 * ROWS_PER,), jnp.float32),
            pltpu.VMEM((ITEM_PER_W,), jnp.int32),
            pltpu.VMEM((ITEM_PER_W, ITEM_D), jnp.float32),
            pltpu.SemaphoreType.DMA,
            pltpu.SemaphoreType.DMA,
            pltpu.SemaphoreType.DMA,
        ],
        compiler_params=pltpu.CompilerParams(needs_layout_passes=False,
                                             use_tc_tiling_on_sc=False),
    )
    def sc_kernel(sdet_hbm, mdet_hbm, items_hbm, emb_hbm,
                  sfT_hbm, mfT_hbm, irows_hbm,
                  deta_v, detb_v, claim_v, tab_v, idx_v, rows_v,
                  sema, semb, semg):
        c = lax.axis_index("c")
        s = lax.axis_index("s")
        wid = s * 2 + c

        ibase = wid * ITEM_PER_W
        pltpu.sync_copy(items_hbm.at[pl.ds(ibase, ITEM_PER_W)], idx_v)
        gcopy = pltpu.async_copy(emb_hbm.at[idx_v], rows_v, semg)

        part = wid & 15
        lo = part * ROWS_PER

        def claim_init(i, _):
            claim_v[pl.ds(i * 16, 16)] = jnp.full((16,), -1, jnp.int32)
            return 0
        lax.fori_loop(0, ROWS_PER // 16, claim_init, 0)

        def tab_init(i, _):
            tab_v[pl.ds(i * 16, 16)] = jnp.zeros((16,), jnp.float32)
            return 0
        lax.fori_loop(0, 4 * ROWS_PER // 16, tab_init, 0)

        @pl.when(wid < 16)
        def _():
            _scatter_one_table(sdet_hbm, (deta_v, detb_v), (sema, semb),
                               claim_v, tab_v, 5, 4, lo)
            for k in range(4):
                pltpu.sync_copy(tab_v.at[pl.ds(k * ROWS_PER, ROWS_PER)],
                                sfT_hbm.at[part, k])

        @pl.when(wid >= 16)
        def _():
            _scatter_one_table(mdet_hbm, (deta_v, detb_v), (sema, semb),
                               claim_v, tab_v, 3, 2, lo)
            for k in range(2):
                pltpu.sync_copy(tab_v.at[pl.ds(k * ROWS_PER, ROWS_PER)],
                                mfT_hbm.at[part, k])

        gcopy.wait()
        pltpu.sync_copy(rows_v, irows_hbm.at[pl.ds(ibase, ITEM_PER_W)])

    return sc_kernel(screen_det.reshape(-1), minimap_det.reshape(-1),
                     items, item_emb)


def _tc_mlps(char_emb, sfT, mfT, A1s, B1s, bs1, W2s, bs2, A1m, B1m, bm1, W2m, bm2):
    def body(char_ref, sfT_ref, mfT_ref,
             a1s_ref, b1s_ref, bs1_ref, w2s_ref, bs2_ref,
             a1m_ref, b1m_ref, bm1_ref, w2m_ref, bm2_ref,
             so_ref, mo_ref):
        cb = char_ref[...]
        sft = sfT_ref[0]
        mft = mfT_ref[0]
        dn = (((0,), (0,)), ((), ()))
        cs = lax.dot_general(sft, b1s_ref[...], dn,
                             preferred_element_type=jnp.float32)
        h = jnp.dot(cb, a1s_ref[...], preferred_element_type=jnp.float32)
        h = jnp.maximum(h + cs + bs1_ref[...], 0.0)
        so = jnp.dot(h, w2s_ref[...], preferred_element_type=jnp.float32)
        so_ref[...] = (so + bs2_ref[...])[None]
        cm = lax.dot_general(mft, b1m_ref[...], dn,
                             preferred_element_type=jnp.float32)
        hm = jnp.dot(cb, a1m_ref[...], preferred_element_type=jnp.float32)
        hm = jnp.maximum(hm + cm + bm1_ref[...], 0.0)
        mo = jnp.dot(hm, w2m_ref[...], preferred_element_type=jnp.float32)
        mo_ref[...] = (mo + bm2_ref[...])[None]

    full = lambda shape: pl.BlockSpec(shape, lambda i: (0,) * len(shape))
    return pl.pallas_call(
        body,
        grid=(N_PART,),
        in_specs=[
            pl.BlockSpec((ROWS_PER, EMB), lambda i: (i, 0)),
            pl.BlockSpec((1, 4, ROWS_PER), lambda i: (i, 0, 0)),
            pl.BlockSpec((1, 2, ROWS_PER), lambda i: (i, 0, 0)),
            full((EMB, EMB)), full((4, EMB)), full((1, EMB)), full((EMB, EMB)), full((1, EMB)),
            full((EMB, EMB)), full((2, EMB)), full((1, EMB)), full((EMB, EMB)), full((1, EMB)),
        ],
        out_specs=[
            pl.BlockSpec((1, ROWS_PER, EMB), lambda i: (i, 0, 0)),
            pl.BlockSpec((1, ROWS_PER, EMB), lambda i: (i, 0, 0)),
        ],
        out_shape=[
            jax.ShapeDtypeStruct((N_PART, ROWS_PER, EMB), jnp.float32),
            jax.ShapeDtypeStruct((N_PART, ROWS_PER, EMB), jnp.float32),
        ],
    )(char_emb, sfT, mfT, A1s, B1s, bs1, W2s, bs2, A1m, B1m, bm1, W2m, bm2)


_SO_LEN = T_ROWS * EMB
_SO_PAD = N_PART * ROWS_PER * EMB
_IT_LEN = N_ITEMS_GAME * ITEM_D
_N_CHUNK = 7
_SO_CH = _SO_PAD // _N_CHUNK
_IT_CH = _IT_LEN // _N_CHUNK
_SO_TAIL = _SO_LEN - 6 * _SO_CH


def _tc_assemble(cont, so1d, mo1d, ir1d):
    def body(cont_ref, so_ref, mo_ref, ir_ref, out_ref, sem):
        i = pl.program_id(0)

        @pl.when(i == 0)
        def _():
            pltpu.async_copy(cont_ref, out_ref.at[0, pl.ds(0, 512)], sem).wait()

        c3 = pltpu.async_copy(
            ir_ref, out_ref.at[0, pl.ds(512 + 2 * _SO_LEN + i * _IT_CH,
                                        _IT_CH)], sem)

        @pl.when(i < _N_CHUNK - 1)
        def _():
            c1 = pltpu.async_copy(
                so_ref, out_ref.at[0, pl.ds(512 + i * _SO_CH, _SO_CH)], sem)
            c2 = pltpu.async_copy(
                mo_ref, out_ref.at[0, pl.ds(512 + _SO_LEN + i * _SO_CH,
                                            _SO_CH)], sem)
            c1.wait()
            c2.wait()

        @pl.when(i == _N_CHUNK - 1)
        def _():
            c1 = pltpu.async_copy(
                so_ref.at[pl.ds(0, _SO_TAIL)],
                out_ref.at[0, pl.ds(512 + i * _SO_CH, _SO_TAIL)], sem)
            c2 = pltpu.async_copy(
                mo_ref.at[pl.ds(0, _SO_TAIL)],
                out_ref.at[0, pl.ds(512 + _SO_LEN + i * _SO_CH, _SO_TAIL)], sem)
            c1.wait()
            c2.wait()

        c3.wait()

    return pl.pallas_call(
        body,
        grid=(_N_CHUNK,),
        in_specs=[
            pl.BlockSpec((512,), lambda i: (0,)),
            pl.BlockSpec((_SO_CH,), lambda i: (i,)),
            pl.BlockSpec((_SO_CH,), lambda i: (i,)),
            pl.BlockSpec((_IT_CH,), lambda i: (i,)),
        ],
        out_specs=pl.BlockSpec(memory_space=pl.ANY),
        out_shape=jax.ShapeDtypeStruct((1, OUT_N), jnp.float32),
        scratch_shapes=[pltpu.SemaphoreType.DMA],
    )(cont, so1d, mo1d, ir1d)


def kernel(continuous_f, screen_detections, minimap_detections, items, char_emb,
           item_emb, Ws1, bs1, Ws2, bs2, Wm1, bm1, Wm2, bm2):
    screen_detections = screen_detections.astype(jnp.int32)
    minimap_detections = minimap_detections.astype(jnp.int32)
    items = items.astype(jnp.int32)

    sfT, mfT, irows = _sc_irregular(screen_detections, minimap_detections,
                                    items, item_emb)

    A1s = Ws1[:, :EMB].T
    B1s = Ws1[:, EMB:].T
    W2s = Ws2.T
    A1m = Wm1[:, :EMB].T
    B1m = Wm1[:, EMB:].T
    W2m = Wm2.T

    so3, mo3 = _tc_mlps(
        char_emb, sfT, mfT,
        A1s, B1s, bs1.reshape(1, EMB), W2s, bs2.reshape(1, EMB),
        A1m, B1m, bm1.reshape(1, EMB), W2m, bm2.reshape(1, EMB))

    return _tc_assemble(continuous_f, so3.reshape(-1), mo3.reshape(-1),
                        irows.reshape(-1))


# speedup vs baseline: 7.8388x; 1.0543x over previous
import functools

import jax
import jax.numpy as jnp
from jax import lax
from jax.experimental import pallas as pl
from jax.experimental.pallas import tpu as pltpu
from jax.experimental.pallas import tpu_sc as plsc

T_ROWS = 50015
EMB = 128
N_DET = 20000
DET_CHUNK = 2000
N_PART = 16
ROWS_PER = 3136
N_ITEMS_GAME = 28672
ITEM_D = 64
ITEM_PER_W = N_ITEMS_GAME // 32

OUT_N = 512 + 2 * T_ROWS * EMB + N_ITEMS_GAME * ITEM_D
SO_ROW0 = 4
MO_ROW0 = SO_ROW0 + T_ROWS
IT_ROW64 = (512 + 2 * T_ROWS * EMB) // ITEM_D
TAIL = T_ROWS - 15 * ROWS_PER


def _scatter_one_table(det_hbm, bufs, sems, claim_v, tab_v, width, ncols, lo):
    lanes = lax.iota(jnp.int32, 16)
    nch = N_DET // DET_CHUNK

    copies = [None, None]
    copies[0] = pltpu.async_copy(det_hbm.at[pl.ds(0, DET_CHUNK * width)],
                                 bufs[0].at[pl.ds(0, DET_CHUNK * width)],
                                 sems[0])
    for ci in range(nch):
        sl = ci % 2
        copies[sl].wait()
        if ci + 1 < nch:
            nsl = (ci + 1) % 2
            copies[nsl] = pltpu.async_copy(
                det_hbm.at[pl.ds((ci + 1) * DET_CHUNK * width,
                                 DET_CHUNK * width)],
                bufs[nsl].at[pl.ds(0, DET_CHUNK * width)], sems[nsl])
        det_v = bufs[sl]
        c0 = ci * DET_CHUNK

        def vreg_body(v, _):
            row = v * 16 + lanes
            j = c0 + row
            flat = row * width
            sid = plsc.load_gather(det_v, [flat])
            m = (sid >= lo) & (sid < lo + ROWS_PER)
            rel = jnp.where(m, sid - lo, 0)

            plsc.store_scatter(claim_v, [rel], j, mask=m)
            cc0 = plsc.load_gather(claim_v, [rel])

            def wcond(cc):
                return jnp.any(m & (j > cc))

            def wbody(cc):
                plsc.store_scatter(claim_v, [rel], j, mask=m & (j > cc))
                return plsc.load_gather(claim_v, [rel])

            ccf = lax.while_loop(wcond, wbody, cc0)
            win = m & (ccf == j)
            for k in range(ncols):
                dk = plsc.load_gather(det_v, [flat + (k + 1)])
                plsc.store_scatter(tab_v, [rel + k * ROWS_PER],
                                   dk.astype(jnp.float32), mask=win)
            return 0

        lax.fori_loop(0, DET_CHUNK // 16, vreg_body, 0)


def _sc_irregular(screen_det, minimap_det, items, item_emb):
    mesh = plsc.VectorSubcoreMesh(core_axis_name="c", subcore_axis_name="s")

    @functools.partial(
        pl.kernel,
        out_type=(
            jax.ShapeDtypeStruct((N_PART, 4, ROWS_PER), jnp.float32),
            jax.ShapeDtypeStruct((N_PART, 2, ROWS_PER), jnp.float32),
            jax.ShapeDtypeStruct((N_ITEMS_GAME, ITEM_D), jnp.float32),
        ),
        mesh=mesh,
        scratch_types=[
            pltpu.VMEM((DET_CHUNK * 5,), jnp.int32),
            pltpu.VMEM((DET_CHUNK * 5,), jnp.int32),
            pltpu.VMEM((ROWS_PER,), jnp.int32),
            pltpu.VMEM((4 * ROWS_PER,), jnp.float32),
            pltpu.VMEM((ITEM_PER_W,), jnp.int32),
            pltpu.VMEM((ITEM_PER_W, ITEM_D), jnp.float32),
            pltpu.SemaphoreType.DMA,
            pltpu.SemaphoreType.DMA,
            pltpu.SemaphoreType.DMA,
        ],
        compiler_params=pltpu.CompilerParams(needs_layout_passes=False,
                                             use_tc_tiling_on_sc=False),
    )
    def sc_kernel(sdet_hbm, mdet_hbm, items_hbm, emb_hbm,
                  sfT_hbm, mfT_hbm, irows_hbm,
                  deta_v, detb_v, claim_v, tab_v, idx_v, rows_v,
                  sema, semb, semg):
        c = lax.axis_index("c")
        s = lax.axis_index("s")
        wid = s * 2 + c

        ibase = wid * ITEM_PER_W
        pltpu.sync_copy(items_hbm.at[pl.ds(ibase, ITEM_PER_W)], idx_v)
        gcopy = pltpu.async_copy(emb_hbm.at[idx_v], rows_v, semg)

        part = wid & 15
        lo = part * ROWS_PER

        def claim_init(i, _):
            claim_v[pl.ds(i * 16, 16)] = jnp.full((16,), -1, jnp.int32)
            return 0
        lax.fori_loop(0, ROWS_PER // 16, claim_init, 0)

        def tab_init(i, _):
            tab_v[pl.ds(i * 16, 16)] = jnp.zeros((16,), jnp.float32)
            return 0
        lax.fori_loop(0, 4 * ROWS_PER // 16, tab_init, 0)

        @pl.when(wid < 16)
        def _():
            _scatter_one_table(sdet_hbm, (deta_v, detb_v), (sema, semb),
                               claim_v, tab_v, 5, 4, lo)
            for k in range(4):
                pltpu.sync_copy(tab_v.at[pl.ds(k * ROWS_PER, ROWS_PER)],
                                sfT_hbm.at[part, k])

        @pl.when(wid >= 16)
        def _():
            _scatter_one_table(mdet_hbm, (deta_v, detb_v), (sema, semb),
                               claim_v, tab_v, 3, 2, lo)
            for k in range(2):
                pltpu.sync_copy(tab_v.at[pl.ds(k * ROWS_PER, ROWS_PER)],
                                mfT_hbm.at[part, k])

        gcopy.wait()
        pltpu.sync_copy(rows_v, irows_hbm.at[pl.ds(ibase, ITEM_PER_W)])

    return sc_kernel(screen_det.reshape(-1), minimap_det.reshape(-1),
                     items, item_emb)


_SO_LEN = T_ROWS * EMB
_BLK = ROWS_PER * EMB
_TAIL_LEN = TAIL * EMB
_IT_CH = N_ITEMS_GAME * ITEM_D // N_PART


def _tc_fused(cont, char_emb, sfT, mfT, ir1d,
              A1s, B1s, bs1, W2s, bs2, A1m, B1m, bm1, W2m, bm2):
    def body(cont_ref, char_ref, sfT_ref, mfT_ref, ir_ref,
             a1s_ref, b1s_ref, bs1_ref, w2s_ref, bs2_ref,
             a1m_ref, b1m_ref, bm1_ref, w2m_ref, bm2_ref,
             out_ref, so_scr, mo_scr, sem):
        i = pl.program_id(0)
        cb = char_ref[...]
        sft = sfT_ref[0]
        mft = mfT_ref[0]
        dn = (((0,), (0,)), ((), ()))
        cs = lax.dot_general(sft, b1s_ref[...], dn,
                             preferred_element_type=jnp.float32)
        h = jnp.dot(cb, a1s_ref[...], preferred_element_type=jnp.float32)
        h = jnp.maximum(h + cs + bs1_ref[...], 0.0)
        so = jnp.dot(h, w2s_ref[...], preferred_element_type=jnp.float32)
        so_scr[...] = (so + bs2_ref[...]).reshape(_BLK)
        cm = lax.dot_general(mft, b1m_ref[...], dn,
                             preferred_element_type=jnp.float32)
        hm = jnp.dot(cb, a1m_ref[...], preferred_element_type=jnp.float32)
        hm = jnp.maximum(hm + cm + bm1_ref[...], 0.0)
        mo = jnp.dot(hm, w2m_ref[...], preferred_element_type=jnp.float32)
        mo_scr[...] = (mo + bm2_ref[...]).reshape(_BLK)

        @pl.when(i == 0)
        def _():
            pltpu.async_copy(cont_ref, out_ref.at[0, pl.ds(0, 512)], sem).wait()

        c3 = pltpu.async_copy(
            ir_ref, out_ref.at[0, pl.ds(512 + 2 * _SO_LEN + i * _IT_CH,
                                        _IT_CH)], sem)

        @pl.when(i < N_PART - 1)
        def _():
            c1 = pltpu.async_copy(
                so_scr, out_ref.at[0, pl.ds(512 + i * _BLK, _BLK)], sem)
            c2 = pltpu.async_copy(
                mo_scr, out_ref.at[0, pl.ds(512 + _SO_LEN + i * _BLK, _BLK)],
                sem)
            c1.wait()
            c2.wait()

        @pl.when(i == N_PART - 1)
        def _():
            c1 = pltpu.async_copy(
                so_scr.at[pl.ds(0, _TAIL_LEN)],
                out_ref.at[0, pl.ds(512 + i * _BLK, _TAIL_LEN)], sem)
            c2 = pltpu.async_copy(
                mo_scr.at[pl.ds(0, _TAIL_LEN)],
                out_ref.at[0, pl.ds(512 + _SO_LEN + i * _BLK, _TAIL_LEN)], sem)
            c1.wait()
            c2.wait()

        c3.wait()

    full = lambda shape: pl.BlockSpec(shape, lambda i: (0,) * len(shape))
    return pl.pallas_call(
        body,
        grid=(N_PART,),
        in_specs=[
            pl.BlockSpec((512,), lambda i: (0,)),
            pl.BlockSpec((ROWS_PER, EMB), lambda i: (i, 0)),
            pl.BlockSpec((1, 4, ROWS_PER), lambda i: (i, 0, 0)),
            pl.BlockSpec((1, 2, ROWS_PER), lambda i: (i, 0, 0)),
            pl.BlockSpec((_IT_CH,), lambda i: (i,)),
            full((EMB, EMB)), full((4, EMB)), full((1, EMB)), full((EMB, EMB)), full((1, EMB)),
            full((EMB, EMB)), full((2, EMB)), full((1, EMB)), full((EMB, EMB)), full((1, EMB)),
        ],
        out_specs=pl.BlockSpec(memory_space=pl.ANY),
        out_shape=jax.ShapeDtypeStruct((1, OUT_N), jnp.float32),
        scratch_shapes=[
            pltpu.VMEM((_BLK,), jnp.float32),
            pltpu.VMEM((_BLK,), jnp.float32),
            pltpu.SemaphoreType.DMA,
        ],
    )(cont, char_emb, sfT, mfT, ir1d,
      A1s, B1s, bs1, W2s, bs2, A1m, B1m, bm1, W2m, bm2)


def kernel(continuous_f, screen_detections, minimap_detections, items, char_emb,
           item_emb, Ws1, bs1, Ws2, bs2, Wm1, bm1, Wm2, bm2):
    screen_detections = screen_detections.astype(jnp.int32)
    minimap_detections = minimap_detections.astype(jnp.int32)
    items = items.astype(jnp.int32)

    sfT, mfT, irows = _sc_irregular(screen_detections, minimap_detections,
                                    items, item_emb)

    A1s = Ws1[:, :EMB].T
    B1s = Ws1[:, EMB:].T
    W2s = Ws2.T
    A1m = Wm1[:, :EMB].T
    B1m = Wm1[:, EMB:].T
    W2m = Wm2.T

    return _tc_fused(
        continuous_f, char_emb, sfT, mfT, irows.reshape(-1),
        A1s, B1s, bs1.reshape(1, EMB), W2s, bs2.reshape(1, EMB),
        A1m, B1m, bm1.reshape(1, EMB), W2m, bm2.reshape(1, EMB))


# speedup vs baseline: 9.4170x; 1.2013x over previous
import functools

import jax
import jax.numpy as jnp
from jax import lax
from jax.experimental import pallas as pl
from jax.experimental.pallas import tpu as pltpu
from jax.experimental.pallas import tpu_sc as plsc

T_ROWS = 50015
EMB = 128
N_DET = 20000
DET_CHUNK = 2000
N_PART = 16
ROWS_PER = 3136
N_ITEMS_GAME = 28672
ITEM_D = 64
ITEM_PER_W = N_ITEMS_GAME // 32

OUT_N = 512 + 2 * T_ROWS * EMB + N_ITEMS_GAME * ITEM_D
SO_ROW0 = 4
MO_ROW0 = SO_ROW0 + T_ROWS
IT_ROW64 = (512 + 2 * T_ROWS * EMB) // ITEM_D
TAIL = T_ROWS - 15 * ROWS_PER


def _scatter_one_table(det_hbm, bufs, sems, tab_v, width, ncols, lo):
    lanes = lax.iota(jnp.int32, 16)
    nch = N_DET // DET_CHUNK

    copies = [None, None]
    copies[0] = pltpu.async_copy(det_hbm.at[pl.ds(0, DET_CHUNK * width)],
                                 bufs[0].at[pl.ds(0, DET_CHUNK * width)],
                                 sems[0])
    for ci in range(nch):
        sl = ci % 2
        copies[sl].wait()
        if ci + 1 < nch:
            nsl = (ci + 1) % 2
            copies[nsl] = pltpu.async_copy(
                det_hbm.at[pl.ds((ci + 1) * DET_CHUNK * width,
                                 DET_CHUNK * width)],
                bufs[nsl].at[pl.ds(0, DET_CHUNK * width)], sems[nsl])
        det_v = bufs[sl]

        def vreg_body(v, _):
            row = v * 16 + lanes
            flat = row * width
            sid = plsc.load_gather(det_v, [flat])
            m = (sid >= lo) & (sid < lo + ROWS_PER)
            rel = jnp.where(m, sid - lo, 0)
            for k in range(ncols):
                dk = plsc.load_gather(det_v, [flat + (k + 1)])
                plsc.store_scatter(tab_v, [rel + k * ROWS_PER],
                                   dk.astype(jnp.float32), mask=m)
            return 0

        lax.fori_loop(0, DET_CHUNK // 16, vreg_body, 0)


def _sc_irregular(screen_det, minimap_det, items, item_emb):
    mesh = plsc.VectorSubcoreMesh(core_axis_name="c", subcore_axis_name="s")

    @functools.partial(
        pl.kernel,
        out_type=(
            jax.ShapeDtypeStruct((N_PART, 4, ROWS_PER), jnp.float32),
            jax.ShapeDtypeStruct((N_PART, 2, ROWS_PER), jnp.float32),
            jax.ShapeDtypeStruct((N_ITEMS_GAME, ITEM_D), jnp.float32),
        ),
        mesh=mesh,
        scratch_types=[
            pltpu.VMEM((DET_CHUNK * 5,), jnp.int32),
            pltpu.VMEM((DET_CHUNK * 5,), jnp.int32),
            pltpu.VMEM((4 * ROWS_PER,), jnp.float32),
            pltpu.VMEM((ITEM_PER_W,), jnp.int32),
            pltpu.VMEM((ITEM_PER_W, ITEM_D), jnp.float32),
            pltpu.SemaphoreType.DMA,
            pltpu.SemaphoreType.DMA,
            pltpu.SemaphoreType.DMA,
        ],
        compiler_params=pltpu.CompilerParams(needs_layout_passes=False,
                                             use_tc_tiling_on_sc=False),
    )
    def sc_kernel(sdet_hbm, mdet_hbm, items_hbm, emb_hbm,
                  sfT_hbm, mfT_hbm, irows_hbm,
                  deta_v, detb_v, tab_v, idx_v, rows_v,
                  sema, semb, semg):
        c = lax.axis_index("c")
        s = lax.axis_index("s")
        wid = s * 2 + c

        ibase = wid * ITEM_PER_W
        pltpu.sync_copy(items_hbm.at[pl.ds(ibase, ITEM_PER_W)], idx_v)
        gcopy = pltpu.async_copy(emb_hbm.at[idx_v], rows_v, semg)

        part = wid & 15
        lo = part * ROWS_PER

        def tab_init(i, _):
            tab_v[pl.ds(i * 16, 16)] = jnp.zeros((16,), jnp.float32)
            return 0
        lax.fori_loop(0, 4 * ROWS_PER // 16, tab_init, 0)

        @pl.when(wid < 16)
        def _():
            _scatter_one_table(sdet_hbm, (deta_v, detb_v), (sema, semb),
                               tab_v, 5, 4, lo)
            for k in range(4):
                pltpu.sync_copy(tab_v.at[pl.ds(k * ROWS_PER, ROWS_PER)],
                                sfT_hbm.at[part, k])

        @pl.when(wid >= 16)
        def _():
            _scatter_one_table(mdet_hbm, (deta_v, detb_v), (sema, semb),
                               tab_v, 3, 2, lo)
            for k in range(2):
                pltpu.sync_copy(tab_v.at[pl.ds(k * ROWS_PER, ROWS_PER)],
                                mfT_hbm.at[part, k])

        gcopy.wait()
        pltpu.sync_copy(rows_v, irows_hbm.at[pl.ds(ibase, ITEM_PER_W)])

    return sc_kernel(screen_det.reshape(-1), minimap_det.reshape(-1),
                     items, item_emb)


_SO_LEN = T_ROWS * EMB
_BLK = ROWS_PER * EMB
_TAIL_LEN = TAIL * EMB
_IT_CH = N_ITEMS_GAME * ITEM_D // N_PART


def _tc_fused(cont, char_emb, sfT, mfT, ir1d,
              A1s, B1s, bs1, W2s, bs2, A1m, B1m, bm1, W2m, bm2):
    def body(cont_ref, char_ref, sfT_ref, mfT_ref, ir_ref,
             a1s_ref, b1s_ref, bs1_ref, w2s_ref, bs2_ref,
             a1m_ref, b1m_ref, bm1_ref, w2m_ref, bm2_ref,
             out_ref, so_scr, mo_scr, sem):
        i = pl.program_id(0)
        cb = char_ref[...]
        sft = sfT_ref[0]
        mft = mfT_ref[0]
        dn = (((0,), (0,)), ((), ()))
        cs = lax.dot_general(sft, b1s_ref[...], dn,
                             preferred_element_type=jnp.float32)
        h = jnp.dot(cb, a1s_ref[...], preferred_element_type=jnp.float32)
        h = jnp.maximum(h + cs + bs1_ref[...], 0.0)
        so = jnp.dot(h, w2s_ref[...], preferred_element_type=jnp.float32)
        so_scr[...] = (so + bs2_ref[...]).reshape(_BLK)
        cm = lax.dot_general(mft, b1m_ref[...], dn,
                             preferred_element_type=jnp.float32)
        hm = jnp.dot(cb, a1m_ref[...], preferred_element_type=jnp.float32)
        hm = jnp.maximum(hm + cm + bm1_ref[...], 0.0)
        mo = jnp.dot(hm, w2m_ref[...], preferred_element_type=jnp.float32)
        mo_scr[...] = (mo + bm2_ref[...]).reshape(_BLK)

        @pl.when(i == 0)
        def _():
            pltpu.async_copy(cont_ref, out_ref.at[0, pl.ds(0, 512)], sem).wait()

        c3 = pltpu.async_copy(
            ir_ref, out_ref.at[0, pl.ds(512 + 2 * _SO_LEN + i * _IT_CH,
                                        _IT_CH)], sem)

        @pl.when(i < N_PART - 1)
        def _():
            c1 = pltpu.async_copy(
                so_scr, out_ref.at[0, pl.ds(512 + i * _BLK, _BLK)], sem)
            c2 = pltpu.async_copy(
                mo_scr, out_ref.at[0, pl.ds(512 + _SO_LEN + i * _BLK, _BLK)],
                sem)
            c1.wait()
            c2.wait()

        @pl.when(i == N_PART - 1)
        def _():
            c1 = pltpu.async_copy(
                so_scr.at[pl.ds(0, _TAIL_LEN)],
                out_ref.at[0, pl.ds(512 + i * _BLK, _TAIL_LEN)], sem)
            c2 = pltpu.async_copy(
                mo_scr.at[pl.ds(0, _TAIL_LEN)],
                out_ref.at[0, pl.ds(512 + _SO_LEN + i * _BLK, _TAIL_LEN)], sem)
            c1.wait()
            c2.wait()

        c3.wait()

    full = lambda shape: pl.BlockSpec(shape, lambda i: (0,) * len(shape))
    return pl.pallas_call(
        body,
        grid=(N_PART,),
        in_specs=[
            pl.BlockSpec((512,), lambda i: (0,)),
            pl.BlockSpec((ROWS_PER, EMB), lambda i: (i, 0)),
            pl.BlockSpec((1, 4, ROWS_PER), lambda i: (i, 0, 0)),
            pl.BlockSpec((1, 2, ROWS_PER), lambda i: (i, 0, 0)),
            pl.BlockSpec((_IT_CH,), lambda i: (i,)),
            full((EMB, EMB)), full((4, EMB)), full((1, EMB)), full((EMB, EMB)), full((1, EMB)),
            full((EMB, EMB)), full((2, EMB)), full((1, EMB)), full((EMB, EMB)), full((1, EMB)),
        ],
        out_specs=pl.BlockSpec(memory_space=pl.ANY),
        out_shape=jax.ShapeDtypeStruct((1, OUT_N), jnp.float32),
        scratch_shapes=[
            pltpu.VMEM((_BLK,), jnp.float32),
            pltpu.VMEM((_BLK,), jnp.float32),
            pltpu.SemaphoreType.DMA,
        ],
    )(cont, char_emb, sfT, mfT, ir1d,
      A1s, B1s, bs1, W2s, bs2, A1m, B1m, bm1, W2m, bm2)


def kernel(continuous_f, screen_detections, minimap_detections, items, char_emb,
           item_emb, Ws1, bs1, Ws2, bs2, Wm1, bm1, Wm2, bm2):
    screen_detections = screen_detections.astype(jnp.int32)
    minimap_detections = minimap_detections.astype(jnp.int32)
    items = items.astype(jnp.int32)

    sfT, mfT, irows = _sc_irregular(screen_detections, minimap_detections,
                                    items, item_emb)

    A1s = Ws1[:, :EMB].T
    B1s = Ws1[:, EMB:].T
    W2s = Ws2.T
    A1m = Wm1[:, :EMB].T
    B1m = Wm1[:, EMB:].T
    W2m = Wm2.T

    return _tc_fused(
        continuous_f, char_emb, sfT, mfT, irows.reshape(-1),
        A1s, B1s, bs1.reshape(1, EMB), W2s, bs2.reshape(1, EMB),
        A1m, B1m, bm1.reshape(1, EMB), W2m, bm2.reshape(1, EMB))


# speedup vs baseline: 10.1122x; 1.0738x over previous
import functools

import jax
import jax.numpy as jnp
from jax import lax
from jax.experimental import pallas as pl
from jax.experimental.pallas import tpu as pltpu
from jax.experimental.pallas import tpu_sc as plsc

T_ROWS = 50015
EMB = 128
N_DET = 20000
DET_CHUNK = 2000
_UNROLL = 5
N_PART = 16
ROWS_PER = 3136
N_ITEMS_GAME = 28672
ITEM_D = 64
ITEM_PER_W = N_ITEMS_GAME // 32

OUT_N = 512 + 2 * T_ROWS * EMB + N_ITEMS_GAME * ITEM_D
SO_ROW0 = 4
MO_ROW0 = SO_ROW0 + T_ROWS
IT_ROW64 = (512 + 2 * T_ROWS * EMB) // ITEM_D
TAIL = T_ROWS - 15 * ROWS_PER


def _scatter_one_table(det_hbm, bufs, sems, tab_v, width, ncols, lo):
    lanes = lax.iota(jnp.int32, 16)
    nch = N_DET // DET_CHUNK

    copies = [None, None]
    copies[0] = pltpu.async_copy(det_hbm.at[pl.ds(0, DET_CHUNK * width)],
                                 bufs[0].at[pl.ds(0, DET_CHUNK * width)],
                                 sems[0])
    for ci in range(nch):
        sl = ci % 2
        copies[sl].wait()
        if ci + 1 < nch:
            nsl = (ci + 1) % 2
            copies[nsl] = pltpu.async_copy(
                det_hbm.at[pl.ds((ci + 1) * DET_CHUNK * width,
                                 DET_CHUNK * width)],
                bufs[nsl].at[pl.ds(0, DET_CHUNK * width)], sems[nsl])
        det_v = bufs[sl]

        def vreg_body(v, _):
            for u in range(_UNROLL):
                row = (v * _UNROLL + u) * 16 + lanes
                flat = row * width
                sid = plsc.load_gather(det_v, [flat])
                m = (sid >= lo) & (sid < lo + ROWS_PER)
                rel = jnp.where(m, sid - lo, 0)
                for k in range(ncols):
                    dk = plsc.load_gather(det_v, [flat + (k + 1)])
                    plsc.store_scatter(tab_v, [rel + k * ROWS_PER],
                                       dk.astype(jnp.float32), mask=m)
            return 0

        lax.fori_loop(0, DET_CHUNK // 16 // _UNROLL, vreg_body, 0)


def _sc_irregular(screen_det, minimap_det, items, item_emb):
    mesh = plsc.VectorSubcoreMesh(core_axis_name="c", subcore_axis_name="s")

    @functools.partial(
        pl.kernel,
        out_type=(
            jax.ShapeDtypeStruct((N_PART, 4, ROWS_PER), jnp.float32),
            jax.ShapeDtypeStruct((N_PART, 2, ROWS_PER), jnp.float32),
            jax.ShapeDtypeStruct((N_ITEMS_GAME, ITEM_D), jnp.float32),
        ),
        mesh=mesh,
        scratch_types=[
            pltpu.VMEM((DET_CHUNK * 5,), jnp.int32),
            pltpu.VMEM((DET_CHUNK * 5,), jnp.int32),
            pltpu.VMEM((4 * ROWS_PER,), jnp.float32),
            pltpu.VMEM((ITEM_PER_W,), jnp.int32),
            pltpu.VMEM((ITEM_PER_W, ITEM_D), jnp.float32),
            pltpu.SemaphoreType.DMA,
            pltpu.SemaphoreType.DMA,
            pltpu.SemaphoreType.DMA,
        ],
        compiler_params=pltpu.CompilerParams(needs_layout_passes=False,
                                             use_tc_tiling_on_sc=False),
    )
    def sc_kernel(sdet_hbm, mdet_hbm, items_hbm, emb_hbm,
                  sfT_hbm, mfT_hbm, irows_hbm,
                  deta_v, detb_v, tab_v, idx_v, rows_v,
                  sema, semb, semg):
        c = lax.axis_index("c")
        s = lax.axis_index("s")
        wid = s * 2 + c

        ibase = wid * ITEM_PER_W
        pltpu.sync_copy(items_hbm.at[pl.ds(ibase, ITEM_PER_W)], idx_v)
        gcopy = pltpu.async_copy(emb_hbm.at[idx_v], rows_v, semg)

        part = wid & 15
        lo = part * ROWS_PER

        def tab_init(i, _):
            tab_v[pl.ds(i * 16, 16)] = jnp.zeros((16,), jnp.float32)
            return 0
        lax.fori_loop(0, 4 * ROWS_PER // 16, tab_init, 0)

        @pl.when(wid < 16)
        def _():
            _scatter_one_table(sdet_hbm, (deta_v, detb_v), (sema, semb),
                               tab_v, 5, 4, lo)
            for k in range(4):
                pltpu.sync_copy(tab_v.at[pl.ds(k * ROWS_PER, ROWS_PER)],
                                sfT_hbm.at[part, k])

        @pl.when(wid >= 16)
        def _():
            _scatter_one_table(mdet_hbm, (deta_v, detb_v), (sema, semb),
                               tab_v, 3, 2, lo)
            for k in range(2):
                pltpu.sync_copy(tab_v.at[pl.ds(k * ROWS_PER, ROWS_PER)],
                                mfT_hbm.at[part, k])

        gcopy.wait()
        pltpu.sync_copy(rows_v, irows_hbm.at[pl.ds(ibase, ITEM_PER_W)])

    return sc_kernel(screen_det.reshape(-1), minimap_det.reshape(-1),
                     items, item_emb)


_SO_LEN = T_ROWS * EMB
_BLK = ROWS_PER * EMB
_TAIL_LEN = TAIL * EMB
_IT_CH = N_ITEMS_GAME * ITEM_D // N_PART


def _tc_fused(cont, char_emb, sfT, mfT, ir1d,
              A1s, B1s, bs1, W2s, bs2, A1m, B1m, bm1, W2m, bm2):
    def body(cont_ref, char_ref, sfT_ref, mfT_ref, ir_ref,
             a1s_ref, b1s_ref, bs1_ref, w2s_ref, bs2_ref,
             a1m_ref, b1m_ref, bm1_ref, w2m_ref, bm2_ref,
             out_ref, so_scr, mo_scr, sems):
        i = pl.program_id(0)
        cb = char_ref[...]
        sft = sfT_ref[0]
        mft = mfT_ref[0]
        dn = (((0,), (0,)), ((), ()))
        cs = lax.dot_general(sft, b1s_ref[...], dn,
                             preferred_element_type=jnp.float32)
        h = jnp.dot(cb, a1s_ref[...], preferred_element_type=jnp.float32)
        h = jnp.maximum(h + cs + bs1_ref[...], 0.0)
        so = jnp.dot(h, w2s_ref[...], preferred_element_type=jnp.float32)
        cm = lax.dot_general(mft, b1m_ref[...], dn,
                             preferred_element_type=jnp.float32)
        hm = jnp.dot(cb, a1m_ref[...], preferred_element_type=jnp.float32)
        hm = jnp.maximum(hm + cm + bm1_ref[...], 0.0)
        mo = jnp.dot(hm, w2m_ref[...], preferred_element_type=jnp.float32)

        slot = lax.rem(i, 2)
        so_sl = so_scr.at[slot]
        mo_sl = mo_scr.at[slot]

        @pl.when(i >= 2)
        def _():
            pltpu.make_async_copy(
                so_sl, out_ref.at[0, pl.ds(512 + (i - 2) * _BLK, _BLK)],
                sems.at[slot]).wait()
            pltpu.make_async_copy(
                mo_sl,
                out_ref.at[0, pl.ds(512 + _SO_LEN + (i - 2) * _BLK, _BLK)],
                sems.at[2 + slot]).wait()

        so_sl[...] = (so + bs2_ref[...]).reshape(_BLK)
        mo_sl[...] = (mo + bm2_ref[...]).reshape(_BLK)

        @pl.when(i == 0)
        def _():
            pltpu.async_copy(cont_ref, out_ref.at[0, pl.ds(0, 512)],
                             sems.at[4]).wait()

        c3 = pltpu.async_copy(
            ir_ref, out_ref.at[0, pl.ds(512 + 2 * _SO_LEN + i * _IT_CH,
                                        _IT_CH)], sems.at[4])

        @pl.when(i < N_PART - 1)
        def _():
            pltpu.async_copy(
                so_sl, out_ref.at[0, pl.ds(512 + i * _BLK, _BLK)],
                sems.at[slot])
            pltpu.async_copy(
                mo_sl, out_ref.at[0, pl.ds(512 + _SO_LEN + i * _BLK, _BLK)],
                sems.at[2 + slot])

        @pl.when(i == N_PART - 1)
        def _():
            other = 1 - slot
            pltpu.make_async_copy(
                so_scr.at[other],
                out_ref.at[0, pl.ds(512 + (i - 1) * _BLK, _BLK)],
                sems.at[other]).wait()
            pltpu.make_async_copy(
                mo_scr.at[other],
                out_ref.at[0, pl.ds(512 + _SO_LEN + (i - 1) * _BLK, _BLK)],
                sems.at[2 + other]).wait()
            pltpu.async_copy(
                so_sl.at[pl.ds(0, _TAIL_LEN)],
                out_ref.at[0, pl.ds(512 + i * _BLK, _TAIL_LEN)],
                sems.at[slot]).wait()
            pltpu.async_copy(
                mo_sl.at[pl.ds(0, _TAIL_LEN)],
                out_ref.at[0, pl.ds(512 + _SO_LEN + i * _BLK, _TAIL_LEN)],
                sems.at[2 + slot]).wait()

        c3.wait()

    full = lambda shape: pl.BlockSpec(shape, lambda i: (0,) * len(shape))
    return pl.pallas_call(
        body,
        grid=(N_PART,),
        in_specs=[
            pl.BlockSpec((512,), lambda i: (0,)),
            pl.BlockSpec((ROWS_PER, EMB), lambda i: (i, 0)),
            pl.BlockSpec((1, 4, ROWS_PER), lambda i: (i, 0, 0)),
            pl.BlockSpec((1, 2, ROWS_PER), lambda i: (i, 0, 0)),
            pl.BlockSpec((_IT_CH,), lambda i: (i,)),
            full((EMB, EMB)), full((4, EMB)), full((1, EMB)), full((EMB, EMB)), full((1, EMB)),
            full((EMB, EMB)), full((2, EMB)), full((1, EMB)), full((EMB, EMB)), full((1, EMB)),
        ],
        out_specs=pl.BlockSpec(memory_space=pl.ANY),
        out_shape=jax.ShapeDtypeStruct((1, OUT_N), jnp.float32),
        scratch_shapes=[
            pltpu.VMEM((2, _BLK), jnp.float32),
            pltpu.VMEM((2, _BLK), jnp.float32),
            pltpu.SemaphoreType.DMA((5,)),
        ],
    )(cont, char_emb, sfT, mfT, ir1d,
      A1s, B1s, bs1, W2s, bs2, A1m, B1m, bm1, W2m, bm2)


def kernel(continuous_f, screen_detections, minimap_detections, items, char_emb,
           item_emb, Ws1, bs1, Ws2, bs2, Wm1, bm1, Wm2, bm2):
    screen_detections = screen_detections.astype(jnp.int32)
    minimap_detections = minimap_detections.astype(jnp.int32)
    items = items.astype(jnp.int32)

    sfT, mfT, irows = _sc_irregular(screen_detections, minimap_detections,
                                    items, item_emb)

    A1s = Ws1[:, :EMB].T
    B1s = Ws1[:, EMB:].T
    W2s = Ws2.T
    A1m = Wm1[:, :EMB].T
    B1m = Wm1[:, EMB:].T
    W2m = Wm2.T

    return _tc_fused(
        continuous_f, char_emb, sfT, mfT, irows.reshape(-1),
        A1s, B1s, bs1.reshape(1, EMB), W2s, bs2.reshape(1, EMB),
        A1m, B1m, bm1.reshape(1, EMB), W2m, bm2.reshape(1, EMB))
